# Initial kernel scaffold; baseline (speedup 1.0000x reference)
import jax
import jax.numpy as jnp
from jax.experimental import pallas as pl


def kernel(z0_nk, t_n, noise, alpha_bar):
    raise NotImplementedError("write your pallas kernel here")


# speedup vs baseline: 2.4018x; 2.4018x over previous
import functools

import jax
import jax.numpy as jnp
from jax import lax
from jax.experimental import pallas as pl
from jax.experimental.pallas import tpu as pltpu
from jax.experimental.pallas import tpu_sc as plsc


def _sqrt_tables(alpha_bar):
    t = alpha_bar.shape[0]
    pad = 1024 - t
    ab = jnp.concatenate([alpha_bar.astype(jnp.float32),
                          jnp.zeros((pad,), jnp.float32)]).reshape(8, 128)

    def body(a_ref, sa_ref, sb_ref):
        a = a_ref[...]
        sa_ref[...] = jnp.sqrt(a)
        sb_ref[...] = jnp.sqrt(jnp.maximum(1.0 - a, 0.0))

    sa, sb = pl.pallas_call(
        body,
        out_shape=(jax.ShapeDtypeStruct((8, 128), jnp.float32),
                   jax.ShapeDtypeStruct((8, 128), jnp.float32)),
    )(ab)
    return sa.reshape(1024), sb.reshape(1024)


def _sc_scale_add(z0, t_n, noise, sa, sb):
    n, k = z0.shape
    info = plsc.get_sparse_core_info()
    nc, ns, lanes = info.num_cores, info.num_subcores, info.num_lanes
    nw = nc * ns
    rpw = n // nw
    rc = 256 if rpw % 256 == 0 else rpw
    nchunks = rpw // rc
    nvec = k // lanes

    mesh = plsc.VectorSubcoreMesh(core_axis_name="c", subcore_axis_name="s")

    @functools.partial(
        pl.kernel,
        mesh=mesh,
        compiler_params=pltpu.CompilerParams(needs_layout_passes=False),
        out_type=jax.ShapeDtypeStruct((n, k), jnp.float32),
        scratch_types=[
            pltpu.VMEM((rc, k), jnp.float32),
            pltpu.VMEM((rc, k), jnp.float32),
            pltpu.VMEM((rpw,), jnp.int32),
            pltpu.VMEM((1024,), jnp.float32),
            pltpu.VMEM((1024,), jnp.float32),
        ],
    )
    def run(z0_h, t_h, nz_h, sa_h, sb_h, out_h, z0_v, nz_v, t_v, sa_v, sb_v):
        wid = lax.axis_index("s") * nc + lax.axis_index("c")
        base = wid * rpw
        pltpu.sync_copy(t_h.at[pl.ds(base, rpw)], t_v)
        pltpu.sync_copy(sa_h, sa_v)
        pltpu.sync_copy(sb_h, sb_v)

        def chunk(c, carry):
            rb = base + c * rc
            pltpu.sync_copy(z0_h.at[pl.ds(rb, rc)], z0_v)
            pltpu.sync_copy(nz_h.at[pl.ds(rb, rc)], nz_v)

            def row(r, carry2):
                lr = c * rc + r
                idx = jnp.full((lanes,), lr, jnp.int32)
                tb = plsc.load_gather(t_v, [idx])
                ab = plsc.load_gather(sa_v, [tb])
                bb = plsc.load_gather(sb_v, [tb])
                for j in range(nvec):
                    s = pl.ds(j * lanes, lanes)
                    z0_v[r, s] = ab * z0_v[r, s] + bb * nz_v[r, s]
                return carry2

            lax.fori_loop(0, rc, row, 0)
            pltpu.sync_copy(z0_v, out_h.at[pl.ds(rb, rc)])
            return carry

        lax.fori_loop(0, nchunks, chunk, 0)

    return run(z0, t_n, noise, sa, sb)


def kernel(z0_nk, t_n, noise, alpha_bar):
    sa, sb = _sqrt_tables(alpha_bar)
    z_t = _sc_scale_add(z0_nk, t_n.astype(jnp.int32), noise, sa, sb)
    return (z_t, noise)
